# baseline (speedup 1.0000x reference)
import jax
import jax.numpy as jnp
from jax import lax
from jax.experimental import pallas as pl
from jax.experimental.pallas import tpu as pltpu
from jax.experimental.pallas import tpu_sc as plsc

_BINS = 256
_SIGMA = 75.0
_NCH = 6
_CPC = 3
_NPIX = 224 * 224
_NSUB = 16
_NW = 32
_CHUNK = _NPIX // _NSUB
_NVEC = _CHUNK // 16
_NUNITS = 2 * _CPC
_USTRIDE = 288
_HSIZE = _NUNITS * _USTRIDE
_NTOT = _NCH * _NPIX
_UNROLL = 7
_Q = 4096
_SCALE = float(255 * _Q)
_TBL = _CPC * _BINS


def _make_lut():
    import numpy as np
    qf = (np.arange(_Q, dtype=np.float64) + 0.5) / _Q
    h1 = 1.0 / (1.0 + np.exp(_SIGMA * qf))
    e3 = 1.0 / (1.0 + np.exp(_SIGMA * (1.0 - qf)))
    return np.concatenate([h1, e3]).astype(np.float32)


_LUT = _make_lut()


def _hist_body(lut_hbm, ref_hbm, dst_hbm, out_hbm,
               lutv, st0, st1, hist, histb, sm0, sm1):
    c = lax.axis_index("c")
    s = lax.axis_index("s")
    bufs, sems = (st0, st1), (sm0, sm1)

    def mk_copy(u):
        src = ref_hbm if u < _CPC else dst_hbm
        ch = c * _CPC + (u % _CPC)
        base = ch * _NPIX + s * _CHUNK
        return pltpu.make_async_copy(
            src.at[pl.ds(base, _CHUNK)], bufs[u % 2], sems[u % 2])

    mk_copy(0).start()
    pltpu.sync_copy(lut_hbm, lutv)

    def zero_body(i, carry):
        hist[pl.ds(i * 16, 16)] = jnp.zeros((16,), jnp.float32)
        histb[pl.ds(i * 16, 16)] = jnp.zeros((16,), jnp.float32)
        return carry

    lax.fori_loop(0, _HSIZE // 16, zero_body, 0)

    for u in range(_NUNITS):
        if u + 1 < _NUNITS:
            mk_copy(u + 1).start()
        mk_copy(u).wait()
        stage = bufs[u % 2]
        hu_a = hist.at[pl.ds(u * _USTRIDE, _USTRIDE)]
        hu_b = histb.at[pl.ds(u * _USTRIDE, _USTRIDE)]

        def body(i, carry):
            for t in range(_UNROLL):
                hu = hu_a if t % 2 == 0 else hu_b
                v = stage[pl.ds(i * (16 * _UNROLL) + t * 16, 16)]
                yi = (v * _SCALE).astype(jnp.int32)
                k = yi >> 12
                q = yi & (_Q - 1)
                g1 = plsc.load_gather(lutv, [q])
                g3 = plsc.load_gather(lutv, [q + _Q])
                plsc.addupdate_scatter(hu, [k + 15], g1)
                plsc.addupdate_scatter(hu, [k + 16], 1.0 - g1 - g3)
                plsc.addupdate_scatter(hu, [k + 17], g3)
            return carry

        lax.fori_loop(0, _NVEC // _UNROLL, body, 0)

    def merge_body(i, carry):
        sl = pl.ds(i * 16, 16)
        hist[sl] = hist[sl] + histb[sl]
        return carry

    lax.fori_loop(0, _HSIZE // 16, merge_body, 0)
    pltpu.sync_copy(
        hist, out_hbm.at[pl.ds((c * _NSUB + s) * _HSIZE, _HSIZE)])


def _loss_body(parts_hbm, dst_hbm, out_hbm,
               pbuf, hist, crbuf, tblv, st0, st1, accv, sm0, sm1):
    c = lax.axis_index("c")
    s = lax.axis_index("s")
    w = s * 2 + c
    bufs, sems = (st0, st1), (sm0, sm1)

    def mk_copy(cl):
        ch = c * _CPC + cl
        base = ch * _NPIX + s * _CHUNK
        return pltpu.make_async_copy(
            dst_hbm.at[pl.ds(base, _CHUNK)], bufs[cl % 2], sems[cl % 2])

    mk_copy(0).start()
    pltpu.sync_copy(parts_hbm.at[pl.ds(c * (_NSUB * _HSIZE), _NSUB * _HSIZE)],
                    pbuf)

    def red_body(j, carry):
        tot = pbuf[pl.ds(j * 16, 16)]
        for r in range(1, _NSUB):
            tot = tot + pbuf[pl.ds(r * _HSIZE + j * 16, 16)]
        hist[pl.ds(j * 16, 16)] = tot
        return carry

    lax.fori_loop(0, _HSIZE // 16, red_body, 0)

    def cdf_unit(u, carry):
        def sum_body(t, tot):
            return tot + hist[pl.ds(u * _USTRIDE + 16 + 16 * t, 16)]

        tot = lax.fori_loop(0, 16, sum_body,
                            jnp.zeros((16,), jnp.float32))
        norm = jnp.maximum(jnp.sum(tot), 1e-12)

        def cs_body(t, cy):
            hn = hist[pl.ds(u * _USTRIDE + 16 + 16 * t, 16)] / norm
            crbuf[pl.ds(u * _BINS + 16 * t, 16)] = plsc.cumsum(hn) + cy
            return cy + jnp.sum(hn)

        lax.fori_loop(0, 16, cs_body, jnp.float32(0.0))
        return carry

    lax.fori_loop(0, _NUNITS, cdf_unit, 0)

    def tbl_task(t_id, carry):
        cl = t_id // 16
        jv = t_id % 16
        v = crbuf[pl.ds((_CPC + cl) * _BINS + jv * 16, 16)]
        lo = jnp.zeros((16,), jnp.int32)
        for step in (128, 64, 32, 16, 8, 4, 2, 1):
            cand = lo + step
            g = plsc.load_gather(crbuf, [cl * _BINS + cand - 1])
            lo = jnp.where(g < v, cand, lo)
        val = jnp.maximum(255.0 - lo.astype(jnp.float32), 0.0) / 255.0
        tblv[pl.ds(cl * _BINS + jv * 16, 16)] = val
        return carry

    lax.fori_loop(0, _CPC * 16, tbl_task, 0)

    acc = jnp.zeros((16,), jnp.float32)
    for cl in range(_CPC):
        if cl + 1 < _CPC:
            mk_copy(cl + 1).start()
        mk_copy(cl).wait()
        stage = bufs[cl % 2]

        def body3(i, ac):
            for t in range(_UNROLL):
                v = stage[pl.ds(i * (16 * _UNROLL) + t * 16, 16)]
                idx = jnp.clip((v * 255.0).astype(jnp.int32), 0, _BINS - 1)
                tv = plsc.load_gather(tblv, [idx + cl * _BINS])
                ac = ac + jnp.abs(v - tv)
            return ac

        acc = lax.fori_loop(0, _NVEC // _UNROLL, body3, acc)
    accv[...] = acc
    pltpu.sync_copy(accv, out_hbm.at[pl.ds(w * 16, 16)])


def kernel(ref, dst):
    rf = ref.reshape(-1)
    df = dst.reshape(-1)
    mesh = plsc.VectorSubcoreMesh(core_axis_name="c", subcore_axis_name="s")
    parts = pl.kernel(
        _hist_body,
        out_type=jax.ShapeDtypeStruct((_NW * _HSIZE,), jnp.float32),
        mesh=mesh,
        scratch_types=[
            pltpu.VMEM((2 * _Q,), jnp.float32),
            pltpu.VMEM((_CHUNK,), jnp.float32),
            pltpu.VMEM((_CHUNK,), jnp.float32),
            pltpu.VMEM((_HSIZE,), jnp.float32),
            pltpu.VMEM((_HSIZE,), jnp.float32),
            pltpu.SemaphoreType.DMA,
            pltpu.SemaphoreType.DMA,
        ],
        compiler_params=pltpu.CompilerParams(needs_layout_passes=False),
    )(jnp.asarray(_LUT), rf, df)
    sums = pl.kernel(
        _loss_body,
        out_type=jax.ShapeDtypeStruct((_NW * 16,), jnp.float32),
        mesh=mesh,
        scratch_types=[
            pltpu.VMEM((_NSUB * _HSIZE,), jnp.float32),
            pltpu.VMEM((_HSIZE,), jnp.float32),
            pltpu.VMEM((_NUNITS * _BINS,), jnp.float32),
            pltpu.VMEM((_TBL,), jnp.float32),
            pltpu.VMEM((_CHUNK,), jnp.float32),
            pltpu.VMEM((_CHUNK,), jnp.float32),
            pltpu.VMEM((16,), jnp.float32),
            pltpu.SemaphoreType.DMA,
            pltpu.SemaphoreType.DMA,
        ],
        compiler_params=pltpu.CompilerParams(needs_layout_passes=False),
    )(parts, df)
    return jnp.sum(sums) / float(_NTOT)


# speedup vs baseline: 1.1204x; 1.1204x over previous
import jax
import jax.numpy as jnp
from jax import lax
from jax.experimental import pallas as pl
from jax.experimental.pallas import tpu as pltpu
from jax.experimental.pallas import tpu_sc as plsc

_BINS = 256
_SIGMA = 75.0
_NCH = 6
_CPC = 3
_NPIX = 224 * 224
_NSUB = 16
_NW = 32
_CHUNK = _NPIX // _NSUB
_NVEC = _CHUNK // 16
_NUNITS = 2 * _CPC
_USTRIDE = 288
_HSIZE = _NUNITS * _USTRIDE
_NTOT = _NCH * _NPIX
_UNROLL = 7
_Q = 4096
_SCALE = float(255 * _Q)
_TBL = _CPC * _BINS


def _make_lut():
    import numpy as np
    qf = (np.arange(_Q, dtype=np.float64) + 0.5) / _Q
    h1 = 1.0 / (1.0 + np.exp(_SIGMA * qf))
    e3 = 1.0 / (1.0 + np.exp(_SIGMA * (1.0 - qf)))
    return np.concatenate([h1, e3]).astype(np.float32)


_LUT = _make_lut()


def _hist_body(lut_hbm, ref_hbm, dst_hbm, out_hbm,
               lutv, st0, st1, hist, sm0, sm1):
    c = lax.axis_index("c")
    s = lax.axis_index("s")
    bufs, sems = (st0, st1), (sm0, sm1)

    def mk_copy(u):
        src = ref_hbm if u < _CPC else dst_hbm
        ch = c * _CPC + (u % _CPC)
        base = ch * _NPIX + s * _CHUNK
        return pltpu.make_async_copy(
            src.at[pl.ds(base, _CHUNK)], bufs[u % 2], sems[u % 2])

    mk_copy(0).start()
    pltpu.sync_copy(lut_hbm, lutv)

    def zero_body(i, carry):
        hist[pl.ds(i * 16, 16)] = jnp.zeros((16,), jnp.float32)
        return carry

    lax.fori_loop(0, _HSIZE // 16, zero_body, 0)

    for u in range(_NUNITS):
        if u + 1 < _NUNITS:
            mk_copy(u + 1).start()
        mk_copy(u).wait()
        stage = bufs[u % 2]
        hu = hist.at[pl.ds(u * _USTRIDE, _USTRIDE)]

        def body(i, carry):
            for t in range(_UNROLL):
                v = stage[pl.ds(i * (16 * _UNROLL) + t * 16, 16)]
                yi = (v * _SCALE).astype(jnp.int32)
                k = yi >> 12
                q = yi & (_Q - 1)
                g1 = plsc.load_gather(lutv, [q])
                g3 = plsc.load_gather(lutv, [q + _Q])
                plsc.addupdate_scatter(hu, [k + 15], g1)
                plsc.addupdate_scatter(hu, [k + 16], 1.0 - g1 - g3)
                plsc.addupdate_scatter(hu, [k + 17], g3)
            return carry

        lax.fori_loop(0, _NVEC // _UNROLL, body, 0)

    pltpu.sync_copy(
        hist, out_hbm.at[pl.ds((c * _NSUB + s) * _HSIZE, _HSIZE)])


def _table_body(h_ref, out_ref):
    h = jnp.sum(h_ref[...], axis=1)
    hb = h.reshape(2 * _NUNITS, _USTRIDE)[:, 16:272]
    norm = jnp.maximum(jnp.sum(jnp.abs(hb), axis=1, keepdims=True), 1e-12)
    hn = hb / norm
    ii = lax.broadcasted_iota(jnp.int32, (_BINS, _BINS), 0)
    jj = lax.broadcasted_iota(jnp.int32, (_BINS, _BINS), 1)
    tri = jnp.where(ii <= jj, 1.0, 0.0)
    cdf = lax.dot_general(hn, tri, (((1,), (0,)), ((), ())),
                          preferred_element_type=jnp.float32,
                          precision=lax.Precision.HIGHEST)
    cr = jnp.concatenate([cdf[0:3], cdf[6:9]], axis=0)
    cd = jnp.concatenate([cdf[3:6], cdf[9:12]], axis=0)
    cnt = jnp.sum(jnp.where(cr[:, :, None] - cd[:, None, :] >= 0.0, 1.0, 0.0),
                  axis=1)
    out_ref[...] = jnp.clip(cnt - 1.0, 0.0, 255.0) / 255.0


def _loss_body(tbl_hbm, dst_hbm, out_hbm, tblv, st0, st1, accv, sm0, sm1):
    c = lax.axis_index("c")
    s = lax.axis_index("s")
    w = s * 2 + c
    bufs, sems = (st0, st1), (sm0, sm1)

    def mk_copy(cl):
        ch = c * _CPC + cl
        base = ch * _NPIX + s * _CHUNK
        return pltpu.make_async_copy(
            dst_hbm.at[pl.ds(base, _CHUNK)], bufs[cl % 2], sems[cl % 2])

    mk_copy(0).start()
    pltpu.sync_copy(tbl_hbm.at[pl.ds(c * _TBL, _TBL)], tblv)

    acc = jnp.zeros((16,), jnp.float32)
    for cl in range(_CPC):
        if cl + 1 < _CPC:
            mk_copy(cl + 1).start()
        mk_copy(cl).wait()
        stage = bufs[cl % 2]

        def body3(i, ac):
            for t in range(_UNROLL):
                v = stage[pl.ds(i * (16 * _UNROLL) + t * 16, 16)]
                idx = jnp.clip((v * 255.0).astype(jnp.int32), 0, _BINS - 1)
                tv = plsc.load_gather(tblv, [idx + cl * _BINS])
                ac = ac + jnp.abs(v - tv)
            return ac

        acc = lax.fori_loop(0, _NVEC // _UNROLL, body3, acc)
    accv[...] = acc
    pltpu.sync_copy(accv, out_hbm.at[pl.ds(w * 16, 16)])


def kernel(ref, dst):
    rf = ref.reshape(-1)
    df = dst.reshape(-1)
    mesh = plsc.VectorSubcoreMesh(core_axis_name="c", subcore_axis_name="s")
    parts = pl.kernel(
        _hist_body,
        out_type=jax.ShapeDtypeStruct((_NW * _HSIZE,), jnp.float32),
        mesh=mesh,
        scratch_types=[
            pltpu.VMEM((2 * _Q,), jnp.float32),
            pltpu.VMEM((_CHUNK,), jnp.float32),
            pltpu.VMEM((_CHUNK,), jnp.float32),
            pltpu.VMEM((_HSIZE,), jnp.float32),
            pltpu.SemaphoreType.DMA,
            pltpu.SemaphoreType.DMA,
        ],
        compiler_params=pltpu.CompilerParams(needs_layout_passes=False),
    )(jnp.asarray(_LUT), rf, df)
    table = pl.pallas_call(
        _table_body,
        out_shape=jax.ShapeDtypeStruct((_NCH, _BINS), jnp.float32),
    )(parts.reshape(2, _NSUB, _NUNITS, _USTRIDE))
    sums = pl.kernel(
        _loss_body,
        out_type=jax.ShapeDtypeStruct((_NW * 16,), jnp.float32),
        mesh=mesh,
        scratch_types=[
            pltpu.VMEM((_TBL,), jnp.float32),
            pltpu.VMEM((_CHUNK,), jnp.float32),
            pltpu.VMEM((_CHUNK,), jnp.float32),
            pltpu.VMEM((16,), jnp.float32),
            pltpu.SemaphoreType.DMA,
            pltpu.SemaphoreType.DMA,
        ],
        compiler_params=pltpu.CompilerParams(needs_layout_passes=False),
    )(table.reshape(-1), df)
    return jnp.sum(sums) / float(_NTOT)
